# baseline (speedup 1.0000x reference)
import jax
import jax.numpy as jnp
from jax import lax
from jax.experimental import pallas as pl
from jax.experimental.pallas import tpu as pltpu
from jax.experimental.pallas import tpu_sc as plsc

NC = 2
NS = 16
NW = NC * NS
K = 80


def _tile_rows(n_p):
  rpt8 = (n_p // NS) // 8 * 8
  rem = n_p - NS * rpt8
  return rpt8, rem


def _make_agg(n_p, f, c_chunks, mode):
  rpt8, rem = _tile_rows(n_p)
  cpw = c_chunks * K
  mesh = plsc.VectorSubcoreMesh(core_axis_name="c", subcore_axis_name="s")
  out_type = [jax.ShapeDtypeStruct((n_p, f), jnp.float32),
              jax.ShapeDtypeStruct((n_p, f), jnp.float32)]
  scratch = [pltpu.VMEM((cpw,), jnp.int32),
             pltpu.VMEM((K, f), jnp.float32),
             pltpu.VMEM_SHARED((n_p, f), jnp.float32)]
  if mode == "feat":
    scratch += [pltpu.VMEM((cpw,), jnp.int32),
                pltpu.VMEM((K, f), jnp.float32),
                pltpu.VMEM((K, f), jnp.float32),
                pltpu.SemaphoreType.DMA,
                pltpu.SemaphoreType.DMA]
  scratch.append(pltpu.SemaphoreType.DMA)

  def body(*refs):
    if mode == "feat":
      (x_hbm, ei_hbm, z_hbm, p0, p1,
       row_v, rows_v, acc, col_v, rows_v1, rows_v2,
       sem0, sem1, sem2) = refs
    else:
      (ones_hbm, ei_hbm, z_hbm, p0, p1, row_v, rows_v, acc, sem0) = refs
    cid = lax.axis_index("c")
    sid = lax.axis_index("s")
    wid = sid * NC + cid
    sl = pl.ds(sid * rpt8, rpt8)
    tail = pl.ds(NS * rpt8, rem)
    pltpu.sync_copy(ei_hbm.at[0, pl.ds(wid * cpw, cpw)], row_v)
    if mode == "feat":
      pltpu.sync_copy(ei_hbm.at[1, pl.ds(wid * cpw, cpw)], col_v)
    else:
      pltpu.sync_copy(ones_hbm, rows_v)
    base = sid * rpt8
    for j in range(rpt8 // K):
      pltpu.sync_copy(z_hbm, acc.at[pl.ds(base + j * K, K)])
    zr = rpt8 % K
    if zr:
      pltpu.sync_copy(z_hbm.at[pl.ds(0, zr)],
                      acc.at[pl.ds(base + (rpt8 // K) * K, zr)])
    if rem:
      @pl.when(sid == 0)
      def _():
        pltpu.sync_copy(z_hbm.at[pl.ds(0, rem)], acc.at[tail])
    plsc.subcore_barrier()

    if mode == "feat":
      assert c_chunks >= 3
      bufs = (rows_v, rows_v1, rows_v2)
      sems = (sem0, sem1, sem2)

      def gather(c, b):
        idx = col_v.at[pl.ds(c * K, K)]
        return pltpu.make_async_copy(x_hbm.at[idx], bufs[b], sems[b])

      def scat(c, b):
        idx = row_v.at[pl.ds(c * K, K)]
        pltpu.sync_copy(bufs[b], acc.at[idx], add=True)

      for b in range(3):
        gather(b, b).start()
      trips = c_chunks // 3

      def trip(i, carry):
        c0 = 3 * i
        for k in range(3):
          gather(c0 + k, k).wait()
          scat(c0 + k, k)
          gather(c0 + 3 + k, k).start()
        return carry

      lax.fori_loop(0, trips - 1, trip, 0)
      for c in range(3 * (trips - 1), c_chunks):
        b = c % 3
        gather(c, b).wait()
        scat(c, b)
        if c + 3 < c_chunks:
          gather(c + 3, b).start()
    else:
      def scat_d(c):
        idx = row_v.at[pl.ds(c * K, K)]
        return pltpu.make_async_copy(rows_v, acc.at[idx], sem0)

      def chunk(c, carry):
        scat_d(c).start(add=True)
        return carry

      lax.fori_loop(0, c_chunks, chunk, 0)

      def drain(c, carry):
        scat_d(c).wait()
        return carry

      lax.fori_loop(0, c_chunks, drain, 0)
    plsc.subcore_barrier()

    @pl.when(cid == 0)
    def _():
      pltpu.sync_copy(acc.at[sl], p0.at[sl])
      if rem:
        @pl.when(sid == 0)
        def _():
          pltpu.sync_copy(acc.at[tail], p0.at[tail])

    @pl.when(cid == 1)
    def _():
      pltpu.sync_copy(acc.at[sl], p1.at[sl])
      if rem:
        @pl.when(sid == 0)
        def _():
          pltpu.sync_copy(acc.at[tail], p1.at[tail])

  return pl.kernel(
      body, mesh=mesh, out_type=out_type, scratch_types=scratch,
      compiler_params=pltpu.CompilerParams(use_tc_tiling_on_sc=False))


def _mm1_body(p0, p1, x, d0, d1, w1, b1, out):
  deg = d0[:, 0:1] + d1[:, 0:1] + 1.0
  xm = (p0[...] + p1[...] + x[...]) / deg
  y = lax.dot_general(xm, w1[...], (((1,), (1,)), ((), ())),
                      preferred_element_type=jnp.float32)
  out[...] = jnp.maximum(y + b1[...], 0.0)


def _mm2_body(q0, q1, x1, d0, d1, w2, b2, woa, wob, bo, out):
  deg = d0[:, 0:1] + d1[:, 0:1] + 1.0
  xm = (q0[...] + q1[...] + x1[...]) / deg
  x2 = jnp.maximum(
      lax.dot_general(xm, w2[...], (((1,), (1,)), ((), ())),
                      preferred_element_type=jnp.float32) + b2[...], 0.0)
  out[...] = (lax.dot_general(x1[...], woa[...], (((1,), (1,)), ((), ())),
                              preferred_element_type=jnp.float32)
              + lax.dot_general(x2, wob[...], (((1,), (1,)), ((), ())),
                                preferred_element_type=jnp.float32)
              + bo[...])


def _row_spec(bm, f):
  return pl.BlockSpec((bm, f), lambda i: (i, 0))


def _full_spec(shape):
  return pl.BlockSpec(shape, lambda i: tuple(0 for _ in shape))


def kernel(x, edge_index, W1, b1, W2, b2, Wo, bo):
  n, f = x.shape
  e = edge_index.shape[1]
  hid = W1.shape[0]
  out_dim = Wo.shape[0]

  c_chunks = -(-e // (NW * K))
  e_p = c_chunks * NW * K
  pad = e_p - e
  n_p = n if (pad == 0 and n % 8 == 0) else -(-(n + 1) // 8) * 8
  if pad:
    pad_blk = jnp.stack([jnp.full((pad,), n, jnp.int32),
                         jnp.zeros((pad,), jnp.int32)])
    ei_p = jnp.concatenate([edge_index, pad_blk], axis=1)
  else:
    ei_p = edge_index
  z_f = jnp.zeros((K, f), jnp.float32)
  z16 = jnp.zeros((K, 16), jnp.float32)
  ones16 = jnp.ones((K, 16), jnp.float32)

  deg_k = _make_agg(n_p, 16, c_chunks, "deg")
  agg1 = _make_agg(n_p, f, c_chunks, "feat")
  agg2 = _make_agg(n_p, hid, c_chunks, "feat")

  d0, d1 = deg_k(ones16, ei_p, z16)
  p0, p1 = agg1(x, ei_p, z_f)

  bm = 2000 if n % 2000 == 0 else 1000
  nb = n // bm
  b1r = b1.reshape(1, hid)
  b2r = b2.reshape(1, hid)
  bor = bo.reshape(1, out_dim)
  woa = Wo[:, :hid]
  wob = Wo[:, hid:]
  p0t, p1t = p0[:n], p1[:n]
  d0t, d1t = d0[:n], d1[:n]

  x1 = pl.pallas_call(
      _mm1_body,
      grid=(nb,),
      in_specs=[_row_spec(bm, f), _row_spec(bm, f), _row_spec(bm, f),
                _row_spec(bm, 16), _row_spec(bm, 16),
                _full_spec((hid, f)), _full_spec((1, hid))],
      out_specs=_row_spec(bm, hid),
      out_shape=jax.ShapeDtypeStruct((n, hid), jnp.float32),
  )(p0t, p1t, x, d0t, d1t, W1, b1r)

  q0, q1 = agg2(x1, ei_p, z_f)

  node_repr = pl.pallas_call(
      _mm2_body,
      grid=(nb,),
      in_specs=[_row_spec(bm, hid), _row_spec(bm, hid), _row_spec(bm, hid),
                _row_spec(bm, 16), _row_spec(bm, 16),
                _full_spec((hid, hid)), _full_spec((1, hid)),
                _full_spec((out_dim, hid)), _full_spec((out_dim, hid)),
                _full_spec((1, out_dim))],
      out_specs=_row_spec(bm, out_dim),
      out_shape=jax.ShapeDtypeStruct((n, out_dim), jnp.float32),
  )(q0[:n], q1[:n], x1, d0t, d1t, W2, b2r, woa, wob, bor)

  return node_repr


# speedup vs baseline: 1.1248x; 1.1248x over previous
import jax
import jax.numpy as jnp
from jax import lax
from jax.experimental import pallas as pl
from jax.experimental.pallas import tpu as pltpu
from jax.experimental.pallas import tpu_sc as plsc

NC = 2
NS = 16
NW = NC * NS
K = 80


def _tile_rows(n_p):
  rpt8 = (n_p // NS) // 8 * 8
  rem = n_p - NS * rpt8
  return rpt8, rem


def _make_agg(n_p, f, c_chunks, mode):
  rpt8, rem = _tile_rows(n_p)
  cpw = c_chunks * K
  mesh = plsc.VectorSubcoreMesh(core_axis_name="c", subcore_axis_name="s")
  out_type = [jax.ShapeDtypeStruct((n_p, f), jnp.float32),
              jax.ShapeDtypeStruct((n_p, f), jnp.float32)]
  scratch = [pltpu.VMEM((cpw,), jnp.int32),
             pltpu.VMEM((K, f), jnp.float32),
             pltpu.VMEM_SHARED((n_p, f), jnp.float32)]
  if mode == "feat":
    scratch += [pltpu.VMEM((cpw,), jnp.int32),
                pltpu.VMEM((K, f), jnp.float32),
                pltpu.VMEM((K, f), jnp.float32),
                pltpu.SemaphoreType.DMA,
                pltpu.SemaphoreType.DMA]
  scratch.append(pltpu.SemaphoreType.DMA)

  def body(*refs):
    if mode == "feat":
      (x_hbm, ei_hbm, z_hbm, p0, p1,
       row_v, rows_v, acc, col_v, rows_v1, rows_v2,
       sem0, sem1, sem2) = refs
    else:
      (ones_hbm, ei_hbm, z_hbm, p0, p1, row_v, rows_v, acc, sem0) = refs
    cid = lax.axis_index("c")
    sid = lax.axis_index("s")
    wid = sid * NC + cid
    sl = pl.ds(sid * rpt8, rpt8)
    tail = pl.ds(NS * rpt8, rem)
    pltpu.sync_copy(ei_hbm.at[0, pl.ds(wid * cpw, cpw)], row_v)
    if mode == "feat":
      pltpu.sync_copy(ei_hbm.at[1, pl.ds(wid * cpw, cpw)], col_v)
    else:
      pltpu.sync_copy(ones_hbm, rows_v)
    pltpu.sync_copy(z_hbm.at[sl], acc.at[sl])
    if rem:
      @pl.when(sid == 0)
      def _():
        pltpu.sync_copy(z_hbm.at[tail], acc.at[tail])
    plsc.subcore_barrier()

    if mode == "feat":
      assert c_chunks >= 3
      bufs = (rows_v, rows_v1, rows_v2)
      sems = (sem0, sem1, sem2)

      def gather(c, b):
        idx = col_v.at[pl.ds(c * K, K)]
        return pltpu.make_async_copy(x_hbm.at[idx], bufs[b], sems[b])

      def scat(c, b):
        idx = row_v.at[pl.ds(c * K, K)]
        pltpu.sync_copy(bufs[b], acc.at[idx], add=True)

      for b in range(3):
        gather(b, b).start()
      trips = c_chunks // 3

      def trip(i, carry):
        c0 = 3 * i
        for k in range(3):
          gather(c0 + k, k).wait()
          scat(c0 + k, k)
          gather(c0 + 3 + k, k).start()
        return carry

      lax.fori_loop(0, trips - 1, trip, 0)
      for c in range(3 * (trips - 1), c_chunks):
        b = c % 3
        gather(c, b).wait()
        scat(c, b)
        if c + 3 < c_chunks:
          gather(c + 3, b).start()
    else:
      def scat_d(c):
        idx = row_v.at[pl.ds(c * K, K)]
        return pltpu.make_async_copy(rows_v, acc.at[idx], sem0)

      def chunk(c, carry):
        scat_d(c).start(add=True)
        return carry

      lax.fori_loop(0, c_chunks, chunk, 0)

      def drain(c, carry):
        scat_d(c).wait()
        return carry

      lax.fori_loop(0, c_chunks, drain, 0)
    plsc.subcore_barrier()

    @pl.when(cid == 0)
    def _():
      pltpu.sync_copy(acc.at[sl], p0.at[sl])
      if rem:
        @pl.when(sid == 0)
        def _():
          pltpu.sync_copy(acc.at[tail], p0.at[tail])

    @pl.when(cid == 1)
    def _():
      pltpu.sync_copy(acc.at[sl], p1.at[sl])
      if rem:
        @pl.when(sid == 0)
        def _():
          pltpu.sync_copy(acc.at[tail], p1.at[tail])

  return pl.kernel(
      body, mesh=mesh, out_type=out_type, scratch_types=scratch,
      compiler_params=pltpu.CompilerParams(use_tc_tiling_on_sc=False))


def _mm1_body(p0, p1, x, d0, d1, w1, b1, out):
  deg = d0[:, 0:1] + d1[:, 0:1] + 1.0
  xm = (p0[...] + p1[...] + x[...]) / deg
  y = lax.dot_general(xm, w1[...], (((1,), (1,)), ((), ())),
                      preferred_element_type=jnp.float32)
  out[...] = jnp.maximum(y + b1[...], 0.0)


def _mm2_body(q0, q1, x1, d0, d1, w2, b2, woa, wob, bo, out):
  deg = d0[:, 0:1] + d1[:, 0:1] + 1.0
  xm = (q0[...] + q1[...] + x1[...]) / deg
  x2 = jnp.maximum(
      lax.dot_general(xm, w2[...], (((1,), (1,)), ((), ())),
                      preferred_element_type=jnp.float32) + b2[...], 0.0)
  out[...] = (lax.dot_general(x1[...], woa[...], (((1,), (1,)), ((), ())),
                              preferred_element_type=jnp.float32)
              + lax.dot_general(x2, wob[...], (((1,), (1,)), ((), ())),
                                preferred_element_type=jnp.float32)
              + bo[...])


def _row_spec(bm, f):
  return pl.BlockSpec((bm, f), lambda i: (i, 0))


def _full_spec(shape):
  return pl.BlockSpec(shape, lambda i: tuple(0 for _ in shape))


def kernel(x, edge_index, W1, b1, W2, b2, Wo, bo):
  n, f = x.shape
  e = edge_index.shape[1]
  hid = W1.shape[0]
  out_dim = Wo.shape[0]

  c_chunks = -(-e // (NW * K))
  e_p = c_chunks * NW * K
  pad = e_p - e
  n_p = n if (pad == 0 and n % 8 == 0) else -(-(n + 1) // 8) * 8
  if pad:
    pad_blk = jnp.stack([jnp.full((pad,), n, jnp.int32),
                         jnp.zeros((pad,), jnp.int32)])
    ei_p = jnp.concatenate([edge_index, pad_blk], axis=1)
  else:
    ei_p = edge_index
  z_f = jnp.zeros((n_p, f), jnp.float32)
  z16 = jnp.zeros((n_p, 16), jnp.float32)
  ones16 = jnp.ones((K, 16), jnp.float32)

  deg_k = _make_agg(n_p, 16, c_chunks, "deg")
  agg1 = _make_agg(n_p, f, c_chunks, "feat")
  agg2 = _make_agg(n_p, hid, c_chunks, "feat")

  d0, d1 = deg_k(ones16, ei_p, z16)
  p0, p1 = agg1(x, ei_p, z_f)

  bm = 2000 if n % 2000 == 0 else 1000
  nb = n // bm
  b1r = b1.reshape(1, hid)
  b2r = b2.reshape(1, hid)
  bor = bo.reshape(1, out_dim)
  woa = Wo[:, :hid]
  wob = Wo[:, hid:]
  p0t, p1t = p0[:n], p1[:n]
  d0t, d1t = d0[:n], d1[:n]

  x1 = pl.pallas_call(
      _mm1_body,
      grid=(nb,),
      in_specs=[_row_spec(bm, f), _row_spec(bm, f), _row_spec(bm, f),
                _row_spec(bm, 16), _row_spec(bm, 16),
                _full_spec((hid, f)), _full_spec((1, hid))],
      out_specs=_row_spec(bm, hid),
      out_shape=jax.ShapeDtypeStruct((n, hid), jnp.float32),
  )(p0t, p1t, x, d0t, d1t, W1, b1r)

  q0, q1 = agg2(x1, ei_p, z_f)

  node_repr = pl.pallas_call(
      _mm2_body,
      grid=(nb,),
      in_specs=[_row_spec(bm, hid), _row_spec(bm, hid), _row_spec(bm, hid),
                _row_spec(bm, 16), _row_spec(bm, 16),
                _full_spec((hid, hid)), _full_spec((1, hid)),
                _full_spec((out_dim, hid)), _full_spec((out_dim, hid)),
                _full_spec((1, out_dim))],
      out_specs=_row_spec(bm, out_dim),
      out_shape=jax.ShapeDtypeStruct((n, out_dim), jnp.float32),
  )(q0[:n], q1[:n], x1, d0t, d1t, W2, b2r, woa, wob, bor)

  return node_repr
